# baseline (speedup 1.0000x reference)
import dataclasses
import functools

import jax
import jax.numpy as jnp
from jax import lax
from jax.experimental import pallas as pl
from jax.experimental.pallas import tpu as pltpu
from jax.experimental.pallas import tpu_sc as plsc

N = 10000
E = 320000
H = 128
HH = H // 2

NC = 2
NS = 16
NW = NC * NS
L = 16

C = 80
ROWS = E // C
RPT_MSG = ROWS // NW
RPT_DEN = ROWS // NS
NP = 10240
NPS = NP // NS

_mesh = plsc.VectorSubcoreMesh(
    core_axis_name="c", subcore_axis_name="s", num_cores=NC, num_subcores=NS
)

_sc_params = pltpu.CompilerParams(
    needs_layout_passes=False, use_tc_tiling_on_sc=False
)


def _tc_layer_body(x_ref, ws_ref, wd_ref, as_ref, ad_ref,
                   lo_ref, hi_ref, es_ref, ed_ref, m_ref):
    x = x_ref[...]
    xs = lax.dot_general(x, ws_ref[...], (((1,), (1,)), ((), ())),
                         preferred_element_type=jnp.float32)
    lo_ref[...] = xs[:, 0:HH]
    hi_ref[...] = xs[:, HH:H]
    es = jnp.dot(xs, as_ref[...])
    ed = jnp.dot(x, jnp.dot(ad_ref[...], wd_ref[...]))
    es_ref[...] = es
    ed_ref[...] = ed
    mm = jnp.max(es) + jnp.max(ed)
    m_ref[...] = jnp.full((L,), jnp.maximum(mm, 0.2 * mm), jnp.float32)


def _tc_layer(x, ws, wd, a_s, a_d):
    return pl.pallas_call(
        _tc_layer_body,
        out_shape=[
            jax.ShapeDtypeStruct((N, HH), jnp.float32),
            jax.ShapeDtypeStruct((N, HH), jnp.float32),
            jax.ShapeDtypeStruct((N,), jnp.float32),
            jax.ShapeDtypeStruct((N,), jnp.float32),
            jax.ShapeDtypeStruct((L,), jnp.float32),
        ],
    )(x, ws, wd, a_s, a_d)


def _combine(lo_ref, hi_ref, b_ref):
    lo = lo_ref[...].reshape(NC, NP, HH)
    hi = hi_ref[...].reshape(NC, NP, HH)
    q_lo = lo[0, 0:N, :] + lo[1, 0:N, :]
    q_hi = hi[0, 0:N, :] + hi[1, 0:N, :]
    return jnp.concatenate([q_lo, q_hi], axis=1) + b_ref[...][None, :]


def _tc_combine_body(lo_ref, hi_ref, b_ref, o_ref):
    o_ref[...] = _combine(lo_ref, hi_ref, b_ref)


def _tc_combine(lo, hi, b):
    return pl.pallas_call(
        _tc_combine_body,
        out_shape=jax.ShapeDtypeStruct((N, H), jnp.float32),
    )(lo, hi, b)


def _tc_relu_body(lo_ref, hi_ref, b_ref, o_ref):
    o_ref[...] = jax.nn.relu(_combine(lo_ref, hi_ref, b_ref))


def _tc_relu_combine(lo, hi, b):
    return pl.pallas_call(
        _tc_relu_body,
        out_shape=jax.ShapeDtypeStruct((N, H), jnp.float32),
    )(lo, hi, b)


@functools.partial(
    pl.kernel,
    out_type=[
        jax.ShapeDtypeStruct((NW, NPS, HH), jnp.float32),
        jax.ShapeDtypeStruct((NW, NPS, HH), jnp.float32),
    ],
    mesh=_mesh,
    scratch_types=[
        pltpu.VMEM((N,), jnp.float32),
        pltpu.VMEM((N,), jnp.float32),
        pltpu.VMEM((N,), jnp.float32),
        pltpu.VMEM((RPT_MSG, C), jnp.int32),
        pltpu.VMEM((RPT_MSG, C), jnp.int32),
        pltpu.VMEM((RPT_MSG, C), jnp.float32),
        pltpu.VMEM((2, C, HH), jnp.float32),
        pltpu.VMEM((32, HH), jnp.float32),
        pltpu.VMEM((2000,), jnp.float32),
        pltpu.VMEM((2, C), jnp.float32),
        pltpu.VMEM((C // L, L), jnp.float32),
        pltpu.VMEM((L,), jnp.float32),
        pltpu.VMEM_SHARED((N,), jnp.float32),
        pltpu.VMEM_SHARED((NP, HH), jnp.float32),
        pltpu.SemaphoreType.DMA,
        pltpu.SemaphoreType.DMA,
    ],
    compiler_params=_sc_params,
)
def _sc_layer(lo_hbm, hi_hbm, src_hbm, dst_hbm, es_hbm, ed_hbm, m_hbm,
              out_lo_hbm, out_hi_hbm,
              es_v, ed_v, den_v, src_v, dst_v, alpha_all, rows_v, zero_v,
              zden_v, ea_row, alpha_v, m_v, den_sh, acc_sh, sem0, sem1):
    sems = (sem0, sem1)
    cid = lax.axis_index("c")
    sid = lax.axis_index("s")
    wid = cid * NS + sid

    zv = jnp.zeros((L,), jnp.float32)

    @pl.loop(0, 32)
    def _(i):
        for k in range(HH // L):
            zero_v[i, pl.ds(k * L, L)] = zv

    @pl.loop(0, 2000 // L)
    def _(i):
        zden_v[pl.ds(i * L, L)] = zv

    pltpu.sync_copy(es_hbm, es_v)
    pltpu.sync_copy(ed_hbm, ed_v)
    pltpu.sync_copy(m_hbm, m_v)

    def zero_acc():
        for k in range(NPS // 32):
            pltpu.sync_copy(zero_v, acc_sh.at[pl.ds(sid * NPS + k * 32, 32)])

    zero_acc()

    @pl.when(sid == 0)
    def _():
        for k in range(N // 2000):
            pltpu.sync_copy(zden_v, den_sh.at[pl.ds(k * 2000, 2000)])

    plsc.subcore_barrier()

    mv = m_v[...]

    def lane_bcast(vec, u):
        idx = lax.full((L, 1), u, jnp.int32)
        return lax.gather(
            vec, idx,
            dimension_numbers=lax.GatherDimensionNumbers(
                offset_dims=(), collapsed_slice_dims=(0,),
                start_index_map=(0,)),
            slice_sizes=(1,),
            mode=lax.GatherScatterMode.PROMISE_IN_BOUNDS)

    def edge_logits(j, g):
        s16 = src_v[j, pl.ds(g * L, L)]
        d16 = dst_v[j, pl.ds(g * L, L)]
        av = plsc.load_gather(es_v, [s16]) + plsc.load_gather(ed_v, [d16])
        av = jnp.maximum(av, 0.2 * av) - mv
        return jnp.exp(av), d16

    def den_row(j, slot, wait_prev):
        if wait_prev:
            pltpu.make_async_copy(ea_row.at[slot], den_sh.at[dst_v.at[j]],
                                  sems[slot]).wait()
        for g in range(C // L):
            ea, _d = edge_logits(j, g)
            ea_row[slot, pl.ds(g * L, L)] = ea
        pltpu.async_copy(ea_row.at[slot], den_sh.at[dst_v.at[j]], sems[slot],
                         add=True)

    for b in range(RPT_DEN // RPT_MSG):
        blk = sid * (RPT_DEN // RPT_MSG) + b
        pltpu.sync_copy(src_hbm.at[blk], src_v)
        pltpu.sync_copy(dst_hbm.at[blk], dst_v)

        den_row(0, 0, False)
        den_row(1, 1, False)

        @pl.loop(2, RPT_MSG - 1, step=2)
        def _(j):
            den_row(j, 0, True)
            den_row(j + 1, 1, True)

        den_row(RPT_MSG - 1, 0, True)
        pltpu.make_async_copy(ea_row.at[0], den_sh.at[dst_v.at[0]],
                              sems[0]).wait()
        pltpu.make_async_copy(ea_row.at[1], den_sh.at[dst_v.at[1]],
                              sems[1]).wait()

    plsc.subcore_barrier()

    pltpu.sync_copy(den_sh, den_v)

    @pl.loop(0, N // L)
    def _(i):
        d = den_v[pl.ds(i * L, L)]
        den_v[pl.ds(i * L, L)] = 1.0 / (d + 1e-16)

    pltpu.sync_copy(src_hbm.at[wid], src_v)
    pltpu.sync_copy(dst_hbm.at[wid], dst_v)

    def msg_chunk(j, slot, data_hbm, first_pass):
        pltpu.make_async_copy(data_hbm.at[src_v.at[j]], rows_v.at[slot],
                              sems[slot]).wait()
        for g in range(C // L):
            if first_pass:
                ea, d16 = edge_logits(j, g)
                a16 = ea * plsc.load_gather(den_v, [d16])
                alpha_v[g, pl.ds(0, L)] = a16
                alpha_all[j, pl.ds(g * L, L)] = a16
            else:
                alpha_v[g, pl.ds(0, L)] = alpha_all[j, pl.ds(g * L, L)]

        @pl.loop(0, C // L)
        def _(g):
            a16 = alpha_v[g, pl.ds(0, L)]
            for u in range(L):
                bc = lane_bcast(a16, u)
                r = g * L + u
                for k in range(HH // L):
                    rows_v[slot, r, pl.ds(k * L, L)] = (
                        rows_v[slot, r, pl.ds(k * L, L)] * bc)

        pltpu.sync_copy(rows_v.at[slot], acc_sh.at[dst_v.at[j]], add=True)

        @pl.when(j + 2 < RPT_MSG)
        def _():
            pltpu.async_copy(data_hbm.at[src_v.at[j + 2]], rows_v.at[slot],
                             sems[slot])

    def msg_pass(data_hbm, out_hbm_half, first_pass):
        pltpu.async_copy(data_hbm.at[src_v.at[0]], rows_v.at[0], sems[0])
        pltpu.async_copy(data_hbm.at[src_v.at[1]], rows_v.at[1], sems[1])

        @pl.loop(0, RPT_MSG - 1, step=2)
        def _(j):
            msg_chunk(j, 0, data_hbm, first_pass)
            msg_chunk(j + 1, 1, data_hbm, first_pass)

        msg_chunk(RPT_MSG - 1, 0, data_hbm, first_pass)

        plsc.subcore_barrier()
        pltpu.sync_copy(acc_sh.at[pl.ds(sid * NPS, NPS)], out_hbm_half.at[wid])

    msg_pass(lo_hbm, out_lo_hbm, True)
    plsc.subcore_barrier()
    zero_acc()
    plsc.subcore_barrier()
    msg_pass(hi_hbm, out_hi_hbm, False)


def kernel(x, edge_index, W1s, W1d, a1s, a1d, b1, W2s, W2d, a2s, a2d, b2):
    src = edge_index[0].reshape(NW, RPT_MSG, C)
    dst = edge_index[1].reshape(NW, RPT_MSG, C)

    lo1, hi1, es1, ed1, m1 = _tc_layer(x, W1s, W1d, a1s, a1d)
    plo1, phi1 = _sc_layer(lo1, hi1, src, dst, es1, ed1, m1)
    h = _tc_relu_combine(plo1, phi1, b1)

    lo2, hi2, es2, ed2, m2 = _tc_layer(h, W2s, W2d, a2s, a2d)
    plo2, phi2 = _sc_layer(lo2, hi2, src, dst, es2, ed2, m2)
    return _tc_combine(plo2, phi2, b2)


# speedup vs baseline: 1.5143x; 1.5143x over previous
import dataclasses
import functools

import jax
import jax.numpy as jnp
from jax import lax
from jax.experimental import pallas as pl
from jax.experimental.pallas import tpu as pltpu
from jax.experimental.pallas import tpu_sc as plsc

N = 10000
E = 320000
H = 128
HH = H // 2

NC = 2
NS = 16
NW = NC * NS
L = 16

C = 80
ROWS = E // C
RPT_MSG = ROWS // NW
RPT_DEN = ROWS // NS
NP = 10240
NPS = NP // NS

_mesh = plsc.VectorSubcoreMesh(
    core_axis_name="c", subcore_axis_name="s", num_cores=NC, num_subcores=NS
)

_sc_params = pltpu.CompilerParams(
    needs_layout_passes=False, use_tc_tiling_on_sc=False
)


def _tc_layer_body(x_ref, ws_ref, wd_ref, as_ref, ad_ref,
                   lo_ref, hi_ref, es_ref, ed_ref, m_ref):
    x = x_ref[...]
    xs = lax.dot_general(x, ws_ref[...], (((1,), (1,)), ((), ())),
                         preferred_element_type=jnp.float32)
    lo_ref[...] = xs[:, 0:HH]
    hi_ref[...] = xs[:, HH:H]
    es = jnp.dot(xs, as_ref[...])
    ed = jnp.dot(x, jnp.dot(ad_ref[...], wd_ref[...]))
    es_ref[...] = es
    ed_ref[...] = ed
    mm = jnp.max(es) + jnp.max(ed)
    m_ref[...] = jnp.full((L,), jnp.maximum(mm, 0.2 * mm), jnp.float32)


def _tc_layer(x, ws, wd, a_s, a_d):
    return pl.pallas_call(
        _tc_layer_body,
        out_shape=[
            jax.ShapeDtypeStruct((N, HH), jnp.float32),
            jax.ShapeDtypeStruct((N, HH), jnp.float32),
            jax.ShapeDtypeStruct((N,), jnp.float32),
            jax.ShapeDtypeStruct((N,), jnp.float32),
            jax.ShapeDtypeStruct((L,), jnp.float32),
        ],
    )(x, ws, wd, a_s, a_d)


def _combine(lo_ref, hi_ref, b_ref):
    lo = lo_ref[...].reshape(NC, NP, HH)
    hi = hi_ref[...].reshape(NC, NP, HH)
    q_lo = lo[0, 0:N, :] + lo[1, 0:N, :]
    q_hi = hi[0, 0:N, :] + hi[1, 0:N, :]
    return jnp.concatenate([q_lo, q_hi], axis=1) + b_ref[...][None, :]


def _tc_combine_body(lo_ref, hi_ref, b_ref, o_ref):
    o_ref[...] = _combine(lo_ref, hi_ref, b_ref)


def _tc_combine(lo, hi, b):
    return pl.pallas_call(
        _tc_combine_body,
        out_shape=jax.ShapeDtypeStruct((N, H), jnp.float32),
    )(lo, hi, b)


def _tc_relu_body(lo_ref, hi_ref, b_ref, o_ref):
    o_ref[...] = jax.nn.relu(_combine(lo_ref, hi_ref, b_ref))


def _tc_relu_combine(lo, hi, b):
    return pl.pallas_call(
        _tc_relu_body,
        out_shape=jax.ShapeDtypeStruct((N, H), jnp.float32),
    )(lo, hi, b)


@functools.partial(
    pl.kernel,
    out_type=[
        jax.ShapeDtypeStruct((NW, NPS, HH), jnp.float32),
        jax.ShapeDtypeStruct((NW, NPS, HH), jnp.float32),
    ],
    mesh=_mesh,
    scratch_types=[
        pltpu.VMEM((N,), jnp.float32),
        pltpu.VMEM((N,), jnp.float32),
        pltpu.VMEM((N,), jnp.float32),
        pltpu.VMEM((RPT_MSG, C), jnp.int32),
        pltpu.VMEM((RPT_MSG, C), jnp.int32),
        pltpu.VMEM((RPT_MSG, C), jnp.float32),
        pltpu.VMEM((2, C, HH), jnp.float32),
        pltpu.VMEM((32, HH), jnp.float32),
        pltpu.VMEM((2000,), jnp.float32),
        pltpu.VMEM((2, C), jnp.float32),
        pltpu.VMEM((C,), jnp.float32),
        pltpu.VMEM((L,), jnp.float32),
        pltpu.VMEM_SHARED((N,), jnp.float32),
        pltpu.VMEM_SHARED((NP, HH), jnp.float32),
        pltpu.SemaphoreType.DMA,
        pltpu.SemaphoreType.DMA,
    ],
    compiler_params=_sc_params,
)
def _sc_layer(lo_hbm, hi_hbm, src_hbm, dst_hbm, es_hbm, ed_hbm, m_hbm,
              out_lo_hbm, out_hi_hbm,
              es_v, ed_v, den_v, src_v, dst_v, alpha_all, rows_v, zero_v,
              zden_v, ea_row, alpha_v, m_v, den_sh, acc_sh, sem0, sem1):
    sems = (sem0, sem1)
    cid = lax.axis_index("c")
    sid = lax.axis_index("s")
    wid = cid * NS + sid

    zv = jnp.zeros((L,), jnp.float32)

    @pl.loop(0, 32)
    def _(i):
        for k in range(HH // L):
            zero_v[i, pl.ds(k * L, L)] = zv

    @pl.loop(0, 2000 // L)
    def _(i):
        zden_v[pl.ds(i * L, L)] = zv

    pltpu.sync_copy(es_hbm, es_v)
    pltpu.sync_copy(ed_hbm, ed_v)
    pltpu.sync_copy(m_hbm, m_v)

    def zero_acc():
        for k in range(NPS // 32):
            pltpu.sync_copy(zero_v, acc_sh.at[pl.ds(sid * NPS + k * 32, 32)])

    zero_acc()

    @pl.when(sid == 0)
    def _():
        for k in range(N // 2000):
            pltpu.sync_copy(zden_v, den_sh.at[pl.ds(k * 2000, 2000)])

    plsc.subcore_barrier()

    mv = m_v[...]

    def edge_logits(j, g):
        s16 = src_v[j, pl.ds(g * L, L)]
        d16 = dst_v[j, pl.ds(g * L, L)]
        av = plsc.load_gather(es_v, [s16]) + plsc.load_gather(ed_v, [d16])
        av = jnp.maximum(av, 0.2 * av) - mv
        return jnp.exp(av), d16

    def den_row(j, slot, wait_prev):
        if wait_prev:
            pltpu.make_async_copy(ea_row.at[slot], den_sh.at[dst_v.at[j]],
                                  sems[slot]).wait()
        for g in range(C // L):
            ea, _d = edge_logits(j, g)
            ea_row[slot, pl.ds(g * L, L)] = ea
        pltpu.async_copy(ea_row.at[slot], den_sh.at[dst_v.at[j]], sems[slot],
                         add=True)

    for b in range(RPT_DEN // RPT_MSG):
        blk = sid * (RPT_DEN // RPT_MSG) + b
        pltpu.sync_copy(src_hbm.at[blk], src_v)
        pltpu.sync_copy(dst_hbm.at[blk], dst_v)

        den_row(0, 0, False)
        den_row(1, 1, False)

        @pl.loop(2, RPT_MSG - 1, step=2)
        def _(j):
            den_row(j, 0, True)
            den_row(j + 1, 1, True)

        den_row(RPT_MSG - 1, 0, True)
        pltpu.make_async_copy(ea_row.at[0], den_sh.at[dst_v.at[0]],
                              sems[0]).wait()
        pltpu.make_async_copy(ea_row.at[1], den_sh.at[dst_v.at[1]],
                              sems[1]).wait()

    plsc.subcore_barrier()

    pltpu.sync_copy(den_sh, den_v)

    @pl.loop(0, N // L)
    def _(i):
        d = den_v[pl.ds(i * L, L)]
        den_v[pl.ds(i * L, L)] = 1.0 / (d + 1e-16)

    pltpu.sync_copy(src_hbm.at[wid], src_v)
    pltpu.sync_copy(dst_hbm.at[wid], dst_v)

    def msg_chunk(j, slot, data_hbm, first_pass):
        pltpu.make_async_copy(data_hbm.at[src_v.at[j]], rows_v.at[slot],
                              sems[slot]).wait()
        for g in range(C // L):
            if first_pass:
                ea, d16 = edge_logits(j, g)
                a16 = ea * plsc.load_gather(den_v, [d16])
                alpha_v[pl.ds(g * L, L)] = a16
                alpha_all[j, pl.ds(g * L, L)] = a16
            else:
                alpha_v[pl.ds(g * L, L)] = alpha_all[j, pl.ds(g * L, L)]

        @pl.loop(0, C, step=2)
        def _(r):
            for u in range(2):
                bc = plsc.load_gather(alpha_v,
                                      [lax.full((L,), r + u, jnp.int32)])
                for k in range(HH // L):
                    rows_v[slot, r + u, pl.ds(k * L, L)] = (
                        rows_v[slot, r + u, pl.ds(k * L, L)] * bc)

        pltpu.sync_copy(rows_v.at[slot], acc_sh.at[dst_v.at[j]], add=True)

        @pl.when(j + 2 < RPT_MSG)
        def _():
            pltpu.async_copy(data_hbm.at[src_v.at[j + 2]], rows_v.at[slot],
                             sems[slot])

    def msg_pass(data_hbm, out_hbm_half, first_pass):
        pltpu.async_copy(data_hbm.at[src_v.at[0]], rows_v.at[0], sems[0])
        pltpu.async_copy(data_hbm.at[src_v.at[1]], rows_v.at[1], sems[1])

        @pl.loop(0, RPT_MSG - 1, step=2)
        def _(j):
            msg_chunk(j, 0, data_hbm, first_pass)
            msg_chunk(j + 1, 1, data_hbm, first_pass)

        msg_chunk(RPT_MSG - 1, 0, data_hbm, first_pass)

        plsc.subcore_barrier()
        pltpu.sync_copy(acc_sh.at[pl.ds(sid * NPS, NPS)], out_hbm_half.at[wid])

    msg_pass(lo_hbm, out_lo_hbm, True)
    plsc.subcore_barrier()
    zero_acc()
    plsc.subcore_barrier()
    msg_pass(hi_hbm, out_hi_hbm, False)


def kernel(x, edge_index, W1s, W1d, a1s, a1d, b1, W2s, W2d, a2s, a2d, b2):
    src = edge_index[0].reshape(NW, RPT_MSG, C)
    dst = edge_index[1].reshape(NW, RPT_MSG, C)

    lo1, hi1, es1, ed1, m1 = _tc_layer(x, W1s, W1d, a1s, a1d)
    plo1, phi1 = _sc_layer(lo1, hi1, src, dst, es1, ed1, m1)
    h = _tc_relu_combine(plo1, phi1, b1)

    lo2, hi2, es2, ed2, m2 = _tc_layer(h, W2s, W2d, a2s, a2d)
    plo2, phi2 = _sc_layer(lo2, hi2, src, dst, es2, ed2, m2)
    return _tc_combine(plo2, phi2, b2)


# speedup vs baseline: 1.7083x; 1.1281x over previous
import dataclasses
import functools

import jax
import jax.numpy as jnp
from jax import lax
from jax.experimental import pallas as pl
from jax.experimental.pallas import tpu as pltpu
from jax.experimental.pallas import tpu_sc as plsc

N = 10000
E = 320000
H = 128
HH = H // 2

NC = 2
NS = 16
NW = NC * NS
L = 16

C = 80
ROWS = E // C
RPT_MSG = ROWS // NW
RPT_DEN = ROWS // NS
NP = 10240
NPS = NP // NS

_mesh = plsc.VectorSubcoreMesh(
    core_axis_name="c", subcore_axis_name="s", num_cores=NC, num_subcores=NS
)

_sc_params = pltpu.CompilerParams(
    needs_layout_passes=False, use_tc_tiling_on_sc=False
)


def _tc_layer_body(x_ref, ws_ref, wd_ref, as_ref, ad_ref,
                   lo_ref, hi_ref, es_ref, ed_ref, m_ref):
    x = x_ref[...]
    xs = lax.dot_general(x, ws_ref[...], (((1,), (1,)), ((), ())),
                         preferred_element_type=jnp.float32)
    lo_ref[...] = xs[:, 0:HH]
    hi_ref[...] = xs[:, HH:H]
    es = jnp.dot(xs, as_ref[...])
    ed = jnp.dot(x, jnp.dot(ad_ref[...], wd_ref[...]))
    es_ref[...] = es
    ed_ref[...] = ed
    mm = jnp.max(es) + jnp.max(ed)
    m_ref[...] = jnp.full((L,), jnp.maximum(mm, 0.2 * mm), jnp.float32)


def _tc_layer(x, ws, wd, a_s, a_d):
    return pl.pallas_call(
        _tc_layer_body,
        out_shape=[
            jax.ShapeDtypeStruct((N, HH), jnp.float32),
            jax.ShapeDtypeStruct((N, HH), jnp.float32),
            jax.ShapeDtypeStruct((N,), jnp.float32),
            jax.ShapeDtypeStruct((N,), jnp.float32),
            jax.ShapeDtypeStruct((L,), jnp.float32),
        ],
    )(x, ws, wd, a_s, a_d)


def _combine(lo_ref, hi_ref, b_ref):
    lo = lo_ref[...].reshape(NC, NP, HH)
    hi = hi_ref[...].reshape(NC, NP, HH)
    q_lo = lo[0, 0:N, :] + lo[1, 0:N, :]
    q_hi = hi[0, 0:N, :] + hi[1, 0:N, :]
    return jnp.concatenate([q_lo, q_hi], axis=1) + b_ref[...][None, :]


def _tc_combine_body(lo_ref, hi_ref, b_ref, o_ref):
    o_ref[...] = _combine(lo_ref, hi_ref, b_ref)


def _tc_combine(lo, hi, b):
    return pl.pallas_call(
        _tc_combine_body,
        out_shape=jax.ShapeDtypeStruct((N, H), jnp.float32),
    )(lo, hi, b)


def _tc_layer2_body(lo_ref, hi_ref, b_ref, ws_ref, wd_ref, as_ref, ad_ref,
                    xlo_ref, xhi_ref, es_ref, ed_ref, m_ref):
    x = jax.nn.relu(_combine(lo_ref, hi_ref, b_ref))
    xs = lax.dot_general(x, ws_ref[...], (((1,), (1,)), ((), ())),
                         preferred_element_type=jnp.float32)
    xlo_ref[...] = xs[:, 0:HH]
    xhi_ref[...] = xs[:, HH:H]
    es = jnp.dot(xs, as_ref[...])
    ed = jnp.dot(x, jnp.dot(ad_ref[...], wd_ref[...]))
    es_ref[...] = es
    ed_ref[...] = ed
    mm = jnp.max(es) + jnp.max(ed)
    m_ref[...] = jnp.full((L,), jnp.maximum(mm, 0.2 * mm), jnp.float32)


def _tc_layer2(lo, hi, b, ws, wd, a_s, a_d):
    return pl.pallas_call(
        _tc_layer2_body,
        out_shape=[
            jax.ShapeDtypeStruct((N, HH), jnp.float32),
            jax.ShapeDtypeStruct((N, HH), jnp.float32),
            jax.ShapeDtypeStruct((N,), jnp.float32),
            jax.ShapeDtypeStruct((N,), jnp.float32),
            jax.ShapeDtypeStruct((L,), jnp.float32),
        ],
    )(lo, hi, b, ws, wd, a_s, a_d)


@functools.partial(
    pl.kernel,
    out_type=[
        jax.ShapeDtypeStruct((NW, NPS, HH), jnp.float32),
        jax.ShapeDtypeStruct((NW, NPS, HH), jnp.float32),
    ],
    mesh=_mesh,
    scratch_types=[
        pltpu.VMEM((N,), jnp.float32),
        pltpu.VMEM((N,), jnp.float32),
        pltpu.VMEM((N,), jnp.float32),
        pltpu.VMEM((RPT_MSG, C), jnp.int32),
        pltpu.VMEM((RPT_MSG, C), jnp.int32),
        pltpu.VMEM((RPT_MSG, C), jnp.float32),
        pltpu.VMEM((3, C, HH), jnp.float32),
        pltpu.VMEM((32, HH), jnp.float32),
        pltpu.VMEM((2000,), jnp.float32),
        pltpu.VMEM((2, C), jnp.float32),
        pltpu.VMEM((C,), jnp.float32),
        pltpu.VMEM((L,), jnp.float32),
        pltpu.VMEM_SHARED((N,), jnp.float32),
        pltpu.VMEM_SHARED((NP, HH), jnp.float32),
        pltpu.SemaphoreType.DMA,
        pltpu.SemaphoreType.DMA,
        pltpu.SemaphoreType.DMA,
        pltpu.SemaphoreType.DMA,
        pltpu.SemaphoreType.DMA,
        pltpu.SemaphoreType.DMA,
    ],
    compiler_params=_sc_params,
)
def _sc_layer(lo_hbm, hi_hbm, src_hbm, dst_hbm, es_hbm, ed_hbm, m_hbm,
              out_lo_hbm, out_hi_hbm,
              es_v, ed_v, den_v, src_v, dst_v, alpha_all, rows_v, zero_v,
              zden_v, ea_row, alpha_v, m_v, den_sh, acc_sh,
              sem0, sem1, sem2, sem3, sem4, sem5):
    sems = (sem0, sem1)
    gsems = (sem0, sem1, sem2)
    ssems = (sem3, sem4, sem5)
    cid = lax.axis_index("c")
    sid = lax.axis_index("s")
    wid = cid * NS + sid

    zv = jnp.zeros((L,), jnp.float32)

    @pl.loop(0, 32)
    def _(i):
        for k in range(HH // L):
            zero_v[i, pl.ds(k * L, L)] = zv

    @pl.loop(0, 2000 // L)
    def _(i):
        zden_v[pl.ds(i * L, L)] = zv

    pltpu.sync_copy(es_hbm, es_v)
    pltpu.sync_copy(ed_hbm, ed_v)
    pltpu.sync_copy(m_hbm, m_v)

    def zero_acc():
        for k in range(NPS // 32):
            pltpu.sync_copy(zero_v, acc_sh.at[pl.ds(sid * NPS + k * 32, 32)])

    zero_acc()

    @pl.when(sid == 0)
    def _():
        for k in range(N // 2000):
            pltpu.sync_copy(zden_v, den_sh.at[pl.ds(k * 2000, 2000)])

    plsc.subcore_barrier()

    mv = m_v[...]

    def edge_logits(j, g):
        s16 = src_v[j, pl.ds(g * L, L)]
        d16 = dst_v[j, pl.ds(g * L, L)]
        av = plsc.load_gather(es_v, [s16]) + plsc.load_gather(ed_v, [d16])
        av = jnp.maximum(av, 0.2 * av) - mv
        return jnp.exp(av), d16

    def den_row(j, slot, wait_prev):
        if wait_prev:
            pltpu.make_async_copy(ea_row.at[slot], den_sh.at[dst_v.at[j]],
                                  sems[slot]).wait()
        for g in range(C // L):
            ea, _d = edge_logits(j, g)
            ea_row[slot, pl.ds(g * L, L)] = ea
        pltpu.async_copy(ea_row.at[slot], den_sh.at[dst_v.at[j]], sems[slot],
                         add=True)

    for b in range(RPT_DEN // RPT_MSG):
        blk = sid * (RPT_DEN // RPT_MSG) + b
        pltpu.sync_copy(src_hbm.at[blk], src_v)
        pltpu.sync_copy(dst_hbm.at[blk], dst_v)

        den_row(0, 0, False)
        den_row(1, 1, False)

        @pl.loop(2, RPT_MSG - 1, step=2)
        def _(j):
            den_row(j, 0, True)
            den_row(j + 1, 1, True)

        den_row(RPT_MSG - 1, 0, True)
        pltpu.make_async_copy(ea_row.at[0], den_sh.at[dst_v.at[0]],
                              sems[0]).wait()
        pltpu.make_async_copy(ea_row.at[1], den_sh.at[dst_v.at[1]],
                              sems[1]).wait()

    plsc.subcore_barrier()

    pltpu.sync_copy(den_sh, den_v)

    @pl.loop(0, N // L)
    def _(i):
        d = den_v[pl.ds(i * L, L)]
        den_v[pl.ds(i * L, L)] = 1.0 / (d + 1e-16)

    pltpu.sync_copy(src_hbm.at[wid], src_v)
    pltpu.sync_copy(dst_hbm.at[wid], dst_v)

    def msg_chunk(j, slot, data_hbm, first_pass, tail):
        pltpu.make_async_copy(data_hbm.at[src_v.at[j]], rows_v.at[slot],
                              gsems[slot]).wait()
        for g in range(C // L):
            if first_pass:
                ea, d16 = edge_logits(j, g)
                a16 = ea * plsc.load_gather(den_v, [d16])
                alpha_v[pl.ds(g * L, L)] = a16
                alpha_all[j, pl.ds(g * L, L)] = a16
            else:
                alpha_v[pl.ds(g * L, L)] = alpha_all[j, pl.ds(g * L, L)]

        @pl.loop(0, C, step=2)
        def _(r):
            for u in range(2):
                bc = plsc.load_gather(alpha_v,
                                      [lax.full((L,), r + u, jnp.int32)])
                for k in range(HH // L):
                    rows_v[slot, r + u, pl.ds(k * L, L)] = (
                        rows_v[slot, r + u, pl.ds(k * L, L)] * bc)

        pltpu.async_copy(rows_v.at[slot], acc_sh.at[dst_v.at[j]], ssems[slot],
                         add=True)

        if not tail:
            nslot = (slot + 2) % 3

            @pl.when(jnp.logical_and(j >= 1, j + 2 < RPT_MSG))
            def _():
                pltpu.make_async_copy(rows_v.at[nslot],
                                      acc_sh.at[dst_v.at[j]],
                                      ssems[nslot]).wait()

            @pl.when(j + 2 < RPT_MSG)
            def _():
                pltpu.async_copy(data_hbm.at[src_v.at[j + 2]],
                                 rows_v.at[nslot], gsems[nslot])

    def msg_pass(data_hbm, out_hbm_half, first_pass):
        pltpu.async_copy(data_hbm.at[src_v.at[0]], rows_v.at[0], gsems[0])
        pltpu.async_copy(data_hbm.at[src_v.at[1]], rows_v.at[1], gsems[1])

        @pl.loop(0, RPT_MSG - 2, step=3)
        def _(j):
            msg_chunk(j, 0, data_hbm, first_pass, False)
            msg_chunk(j + 1, 1, data_hbm, first_pass, False)
            msg_chunk(j + 2, 2, data_hbm, first_pass, False)

        msg_chunk(RPT_MSG - 2, 0, data_hbm, first_pass, True)
        msg_chunk(RPT_MSG - 1, 1, data_hbm, first_pass, True)
        for b in range(3):
            pltpu.make_async_copy(rows_v.at[b], acc_sh.at[dst_v.at[0]],
                                  ssems[b]).wait()

        plsc.subcore_barrier()
        pltpu.sync_copy(acc_sh.at[pl.ds(sid * NPS, NPS)], out_hbm_half.at[wid])

    msg_pass(lo_hbm, out_lo_hbm, True)
    plsc.subcore_barrier()
    zero_acc()
    plsc.subcore_barrier()
    msg_pass(hi_hbm, out_hi_hbm, False)


def kernel(x, edge_index, W1s, W1d, a1s, a1d, b1, W2s, W2d, a2s, a2d, b2):
    src = edge_index[0].reshape(NW, RPT_MSG, C)
    dst = edge_index[1].reshape(NW, RPT_MSG, C)

    lo1, hi1, es1, ed1, m1 = _tc_layer(x, W1s, W1d, a1s, a1d)
    plo1, phi1 = _sc_layer(lo1, hi1, src, dst, es1, ed1, m1)
    lo2, hi2, es2, ed2, m2 = _tc_layer2(plo1, phi1, b1, W2s, W2d, a2s, a2d)
    plo2, phi2 = _sc_layer(lo2, hi2, src, dst, es2, ed2, m2)
    return _tc_combine(plo2, phi2, b2)


# speedup vs baseline: 1.7111x; 1.0017x over previous
import dataclasses
import functools

import jax
import jax.numpy as jnp
from jax import lax
from jax.experimental import pallas as pl
from jax.experimental.pallas import tpu as pltpu
from jax.experimental.pallas import tpu_sc as plsc

N = 10000
E = 320000
H = 128
HH = H // 2

NC = 2
NS = 16
NW = NC * NS
L = 16

C = 80
ROWS = E // C
RPT_MSG = ROWS // NW
RPT_DEN = ROWS // NS
NP = 10240
NPS = NP // NS

_mesh = plsc.VectorSubcoreMesh(
    core_axis_name="c", subcore_axis_name="s", num_cores=NC, num_subcores=NS
)

_sc_params = pltpu.CompilerParams(
    needs_layout_passes=False, use_tc_tiling_on_sc=False
)


def _tc_layer_body(x_ref, ws_ref, wd_ref, as_ref, ad_ref,
                   lo_ref, hi_ref, es_ref, ed_ref, m_ref):
    x = x_ref[...]
    xs = lax.dot_general(x, ws_ref[...], (((1,), (1,)), ((), ())),
                         preferred_element_type=jnp.float32)
    lo_ref[...] = xs[:, 0:HH]
    hi_ref[...] = xs[:, HH:H]
    es = jnp.dot(xs, as_ref[...])
    ed = jnp.dot(x, jnp.dot(ad_ref[...], wd_ref[...]))
    es_ref[...] = es
    ed_ref[...] = ed
    mm = jnp.max(es) + jnp.max(ed)
    m_ref[...] = jnp.full((L,), jnp.maximum(mm, 0.2 * mm), jnp.float32)


def _tc_layer(x, ws, wd, a_s, a_d):
    return pl.pallas_call(
        _tc_layer_body,
        out_shape=[
            jax.ShapeDtypeStruct((N, HH), jnp.float32),
            jax.ShapeDtypeStruct((N, HH), jnp.float32),
            jax.ShapeDtypeStruct((N,), jnp.float32),
            jax.ShapeDtypeStruct((N,), jnp.float32),
            jax.ShapeDtypeStruct((L,), jnp.float32),
        ],
    )(x, ws, wd, a_s, a_d)


def _combine(lo_ref, hi_ref, b_ref):
    lo = lo_ref[...].reshape(NC, NP, HH)
    hi = hi_ref[...].reshape(NC, NP, HH)
    q_lo = lo[0, 0:N, :] + lo[1, 0:N, :]
    q_hi = hi[0, 0:N, :] + hi[1, 0:N, :]
    return jnp.concatenate([q_lo, q_hi], axis=1) + b_ref[...][None, :]


def _tc_combine_body(lo_ref, hi_ref, b_ref, o_ref):
    o_ref[...] = _combine(lo_ref, hi_ref, b_ref)


def _tc_combine(lo, hi, b):
    return pl.pallas_call(
        _tc_combine_body,
        out_shape=jax.ShapeDtypeStruct((N, H), jnp.float32),
    )(lo, hi, b)


def _tc_layer2_body(lo_ref, hi_ref, b_ref, ws_ref, wd_ref, as_ref, ad_ref,
                    xlo_ref, xhi_ref, es_ref, ed_ref, m_ref):
    x = jax.nn.relu(_combine(lo_ref, hi_ref, b_ref))
    xs = lax.dot_general(x, ws_ref[...], (((1,), (1,)), ((), ())),
                         preferred_element_type=jnp.float32)
    xlo_ref[...] = xs[:, 0:HH]
    xhi_ref[...] = xs[:, HH:H]
    es = jnp.dot(xs, as_ref[...])
    ed = jnp.dot(x, jnp.dot(ad_ref[...], wd_ref[...]))
    es_ref[...] = es
    ed_ref[...] = ed
    mm = jnp.max(es) + jnp.max(ed)
    m_ref[...] = jnp.full((L,), jnp.maximum(mm, 0.2 * mm), jnp.float32)


def _tc_layer2(lo, hi, b, ws, wd, a_s, a_d):
    return pl.pallas_call(
        _tc_layer2_body,
        out_shape=[
            jax.ShapeDtypeStruct((N, HH), jnp.float32),
            jax.ShapeDtypeStruct((N, HH), jnp.float32),
            jax.ShapeDtypeStruct((N,), jnp.float32),
            jax.ShapeDtypeStruct((N,), jnp.float32),
            jax.ShapeDtypeStruct((L,), jnp.float32),
        ],
    )(lo, hi, b, ws, wd, a_s, a_d)


@functools.partial(
    pl.kernel,
    out_type=[
        jax.ShapeDtypeStruct((NW, NPS, HH), jnp.float32),
        jax.ShapeDtypeStruct((NW, NPS, HH), jnp.float32),
    ],
    mesh=_mesh,
    scratch_types=[
        pltpu.VMEM((N,), jnp.float32),
        pltpu.VMEM((N,), jnp.float32),
        pltpu.VMEM((N,), jnp.float32),
        pltpu.VMEM((RPT_MSG, C), jnp.int32),
        pltpu.VMEM((RPT_MSG, C), jnp.int32),
        pltpu.VMEM((RPT_MSG, C), jnp.float32),
        pltpu.VMEM((3, C, HH), jnp.float32),
        pltpu.VMEM((32, HH), jnp.float32),
        pltpu.VMEM((2000,), jnp.float32),
        pltpu.VMEM((2, C), jnp.float32),
        pltpu.VMEM((C,), jnp.float32),
        pltpu.VMEM((L,), jnp.float32),
        pltpu.VMEM_SHARED((N,), jnp.float32),
        pltpu.VMEM_SHARED((NP, HH), jnp.float32),
        pltpu.SemaphoreType.DMA,
        pltpu.SemaphoreType.DMA,
        pltpu.SemaphoreType.DMA,
        pltpu.SemaphoreType.DMA,
        pltpu.SemaphoreType.DMA,
        pltpu.SemaphoreType.DMA,
    ],
    compiler_params=_sc_params,
)
def _sc_layer(lo_hbm, hi_hbm, src_hbm, dst_hbm, es_hbm, ed_hbm, m_hbm,
              out_lo_hbm, out_hi_hbm,
              es_v, ed_v, den_v, src_v, dst_v, alpha_all, rows_v, zero_v,
              zden_v, ea_row, alpha_v, m_v, den_sh, acc_sh,
              sem0, sem1, sem2, sem3, sem4, sem5):
    sems = (sem0, sem1)
    gsems = (sem0, sem1, sem2)
    ssems = (sem3, sem4, sem5)
    cid = lax.axis_index("c")
    sid = lax.axis_index("s")
    wid = cid * NS + sid

    zv = jnp.zeros((L,), jnp.float32)

    @pl.loop(0, 32)
    def _(i):
        for k in range(HH // L):
            zero_v[i, pl.ds(k * L, L)] = zv

    @pl.loop(0, 2000 // L)
    def _(i):
        zden_v[pl.ds(i * L, L)] = zv

    pltpu.sync_copy(es_hbm, es_v)
    pltpu.sync_copy(ed_hbm, ed_v)
    pltpu.sync_copy(m_hbm, m_v)

    def zero_acc():
        for k in range(NPS // 32):
            pltpu.sync_copy(zero_v, acc_sh.at[pl.ds(sid * NPS + k * 32, 32)])

    zero_acc()

    @pl.when(sid == 0)
    def _():
        for k in range(N // 2000):
            pltpu.sync_copy(zden_v, den_sh.at[pl.ds(k * 2000, 2000)])

    plsc.subcore_barrier()

    mv = m_v[...]

    def edge_logits(j, g):
        s16 = src_v[j, pl.ds(g * L, L)]
        d16 = dst_v[j, pl.ds(g * L, L)]
        av = plsc.load_gather(es_v, [s16]) + plsc.load_gather(ed_v, [d16])
        av = jnp.maximum(av, 0.2 * av) - mv
        return jnp.exp(av), d16

    def den_row(j, slot, wait_prev):
        if wait_prev:
            pltpu.make_async_copy(ea_row.at[slot], den_sh.at[dst_v.at[j]],
                                  sems[slot]).wait()
        for g in range(C // L):
            ea, _d = edge_logits(j, g)
            ea_row[slot, pl.ds(g * L, L)] = ea
        pltpu.async_copy(ea_row.at[slot], den_sh.at[dst_v.at[j]], sems[slot],
                         add=True)

    for b in range(RPT_DEN // RPT_MSG):
      with jax.named_scope(f"p1_den_{b}"):
        blk = sid * (RPT_DEN // RPT_MSG) + b
        pltpu.sync_copy(src_hbm.at[blk], src_v)
        pltpu.sync_copy(dst_hbm.at[blk], dst_v)

        den_row(0, 0, False)
        den_row(1, 1, False)

        @pl.loop(2, RPT_MSG - 1, step=2)
        def _(j):
            den_row(j, 0, True)
            den_row(j + 1, 1, True)

        den_row(RPT_MSG - 1, 0, True)
        pltpu.make_async_copy(ea_row.at[0], den_sh.at[dst_v.at[0]],
                              sems[0]).wait()
        pltpu.make_async_copy(ea_row.at[1], den_sh.at[dst_v.at[1]],
                              sems[1]).wait()

    plsc.subcore_barrier()

    with jax.named_scope("recip"):
        pltpu.sync_copy(den_sh, den_v)

        @pl.loop(0, N // L)
        def _(i):
            d = den_v[pl.ds(i * L, L)]
            den_v[pl.ds(i * L, L)] = 1.0 / (d + 1e-16)

    pltpu.sync_copy(src_hbm.at[wid], src_v)
    pltpu.sync_copy(dst_hbm.at[wid], dst_v)

    def msg_chunk(j, slot, data_hbm, first_pass, tail):
        pltpu.make_async_copy(data_hbm.at[src_v.at[j]], rows_v.at[slot],
                              gsems[slot]).wait()
        for g in range(C // L):
            if first_pass:
                ea, d16 = edge_logits(j, g)
                a16 = ea * plsc.load_gather(den_v, [d16])
                alpha_v[pl.ds(g * L, L)] = a16
                alpha_all[j, pl.ds(g * L, L)] = a16
            else:
                alpha_v[pl.ds(g * L, L)] = alpha_all[j, pl.ds(g * L, L)]

        @pl.loop(0, C, step=2)
        def _(r):
            for u in range(2):
                bc = plsc.load_gather(alpha_v,
                                      [lax.full((L,), r + u, jnp.int32)])
                for k in range(HH // L):
                    rows_v[slot, r + u, pl.ds(k * L, L)] = (
                        rows_v[slot, r + u, pl.ds(k * L, L)] * bc)

        pltpu.async_copy(rows_v.at[slot], acc_sh.at[dst_v.at[j]], ssems[slot],
                         add=True)

        if not tail:
            nslot = (slot + 2) % 3

            @pl.when(jnp.logical_and(j >= 1, j + 2 < RPT_MSG))
            def _():
                pltpu.make_async_copy(rows_v.at[nslot],
                                      acc_sh.at[dst_v.at[j]],
                                      ssems[nslot]).wait()

            @pl.when(j + 2 < RPT_MSG)
            def _():
                pltpu.async_copy(data_hbm.at[src_v.at[j + 2]],
                                 rows_v.at[nslot], gsems[nslot])

    def msg_pass(data_hbm, out_hbm_half, first_pass):
        pltpu.async_copy(data_hbm.at[src_v.at[0]], rows_v.at[0], gsems[0])
        pltpu.async_copy(data_hbm.at[src_v.at[1]], rows_v.at[1], gsems[1])

        @pl.loop(0, RPT_MSG - 2, step=3)
        def _(j):
            msg_chunk(j, 0, data_hbm, first_pass, False)
            msg_chunk(j + 1, 1, data_hbm, first_pass, False)
            msg_chunk(j + 2, 2, data_hbm, first_pass, False)

        msg_chunk(RPT_MSG - 2, 0, data_hbm, first_pass, True)
        msg_chunk(RPT_MSG - 1, 1, data_hbm, first_pass, True)
        for b in range(3):
            pltpu.make_async_copy(rows_v.at[b], acc_sh.at[dst_v.at[0]],
                                  ssems[b]).wait()

        plsc.subcore_barrier()
        pltpu.sync_copy(acc_sh.at[pl.ds(sid * NPS, NPS)], out_hbm_half.at[wid])

    with jax.named_scope("pass_lo"):
        msg_pass(lo_hbm, out_lo_hbm, True)
    with jax.named_scope("rezero"):
        plsc.subcore_barrier()
        zero_acc()
        plsc.subcore_barrier()
    with jax.named_scope("pass_hi"):
        msg_pass(hi_hbm, out_hi_hbm, False)


def kernel(x, edge_index, W1s, W1d, a1s, a1d, b1, W2s, W2d, a2s, a2d, b2):
    src = edge_index[0].reshape(NW, RPT_MSG, C)
    dst = edge_index[1].reshape(NW, RPT_MSG, C)

    lo1, hi1, es1, ed1, m1 = _tc_layer(x, W1s, W1d, a1s, a1d)
    plo1, phi1 = _sc_layer(lo1, hi1, src, dst, es1, ed1, m1)
    lo2, hi2, es2, ed2, m2 = _tc_layer2(plo1, phi1, b1, W2s, W2d, a2s, a2d)
    plo2, phi2 = _sc_layer(lo2, hi2, src, dst, es2, ed2, m2)
    return _tc_combine(plo2, phi2, b2)
